# initial kernel scaffold (unmeasured)
import jax
import jax.numpy as jnp
from jax import lax
from jax.experimental import pallas as pl
from jax.experimental.pallas import tpu as pltpu

N_DEV = 4
B = 2
SQ = 512
SKV = 512
HQ = 8
DH = 64
D_MODEL = 768
D_QK = HQ * DH
BLK = 64
NEG = -1e9


def kernel(x, Wq, K_ext, V_ext, Wo):
    k2 = K_ext.reshape(B, SKV, D_QK)
    v2 = V_ext.reshape(B, SKV, D_QK)

    def body(x_ref, wq_ref, k_ref, v_ref, wo_ref, out_ref,
             kfull, vfull, ctx, send_k, recv_k, send_v, recv_v):
        me = lax.axis_index("i")
        right = lax.rem(me + 1, N_DEV)

        kfull[me] = k_ref[...]
        vfull[me] = v_ref[...]

        for h in range(N_DEV - 1):
            slot = lax.rem(me - h + N_DEV, N_DEV)
            rk = pltpu.make_async_remote_copy(
                src_ref=kfull.at[slot],
                dst_ref=kfull.at[slot],
                send_sem=send_k.at[h],
                recv_sem=recv_k.at[h],
                device_id=(right,),
                device_id_type=pl.DeviceIdType.MESH,
            )
            rv = pltpu.make_async_remote_copy(
                src_ref=vfull.at[slot],
                dst_ref=vfull.at[slot],
                send_sem=send_v.at[h],
                recv_sem=recv_v.at[h],
                device_id=(right,),
                device_id_type=pl.DeviceIdType.MESH,
            )
            rk.start()
            rv.start()
            rk.wait()
            rv.wait()

        row = lax.broadcasted_iota(jnp.int32, (SQ, N_DEV * SKV), 0)
        col = lax.broadcasted_iota(jnp.int32, (SQ, N_DEV * SKV), 1)
        qb = me * (SQ // BLK) + row // BLK
        kb = col // BLK
        mask = (qb == kb) | (kb == 0) | (lax.rem(qb + kb, 3) == 0)

        for b in range(B):
            qp = jnp.dot(x_ref[b], wq_ref[...],
                         preferred_element_type=jnp.float32)
            for h in range(HQ):
                q = qp[:, h * DH:(h + 1) * DH]
                parts = []
                for o in range(N_DEV):
                    k_o = kfull[o, b][:, h * DH:(h + 1) * DH]
                    parts.append(lax.dot_general(
                        q, k_o, (((1,), (1,)), ((), ())),
                        preferred_element_type=jnp.float32))
                s = jnp.concatenate(parts, axis=1) * 0.125
                s = jnp.where(mask, s, NEG)
                m = jnp.max(s, axis=1, keepdims=True)
                w = jnp.exp(s - m)
                w = w / jnp.sum(w, axis=1, keepdims=True)
                acc = jnp.zeros((SQ, DH), jnp.float32)
                for o in range(N_DEV):
                    v_o = vfull[o, b][:, h * DH:(h + 1) * DH]
                    acc = acc + jnp.dot(
                        w[:, o * SKV:(o + 1) * SKV], v_o,
                        preferred_element_type=jnp.float32)
                ctx[b, :, h * DH:(h + 1) * DH] = acc
            out_ref[b] = jnp.dot(ctx[b], wo_ref[...],
                                 preferred_element_type=jnp.float32)

    return pl.pallas_call(
        body,
        out_shape=jax.ShapeDtypeStruct((B, SQ, D_MODEL), jnp.float32),
        in_specs=[pl.BlockSpec(memory_space=pltpu.VMEM)] * 5,
        out_specs=pl.BlockSpec(memory_space=pltpu.VMEM),
        scratch_shapes=[
            pltpu.VMEM((N_DEV, B, SKV, D_QK), jnp.float32),
            pltpu.VMEM((N_DEV, B, SKV, D_QK), jnp.float32),
            pltpu.VMEM((B, SQ, D_QK), jnp.float32),
            pltpu.SemaphoreType.DMA((N_DEV - 1,)),
            pltpu.SemaphoreType.DMA((N_DEV - 1,)),
            pltpu.SemaphoreType.DMA((N_DEV - 1,)),
            pltpu.SemaphoreType.DMA((N_DEV - 1,)),
        ],
        compiler_params=pltpu.CompilerParams(collective_id=0),
    )(x, Wq, k2, v2, Wo)


# baseline (device time: 192757 ns/iter reference)
import jax
import jax.numpy as jnp
from jax import lax
from jax.experimental import pallas as pl
from jax.experimental.pallas import tpu as pltpu

N_DEV = 4
B = 2
SQ = 512
SKV = 512
HQ = 8
DH = 64
D_MODEL = 768
D_QK = HQ * DH
BLK = 64
NEG = -1e9


def kernel(x, Wq, K_ext, V_ext, Wo):
    k2 = K_ext.reshape(B, SKV, D_QK)
    v2 = V_ext.reshape(B, SKV, D_QK)

    def body(x_ref, wq_ref, k_ref, v_ref, wo_ref, out_ref,
             kfull, vfull, ctx, send_k, recv_k, send_v, recv_v):
        me = lax.axis_index("i")
        right = lax.rem(me + 1, N_DEV)

        kfull[me] = k_ref[...]
        vfull[me] = v_ref[...]

        for h in range(N_DEV - 1):
            slot = lax.rem(me - h + N_DEV, N_DEV)
            rk = pltpu.make_async_remote_copy(
                src_ref=kfull.at[slot],
                dst_ref=kfull.at[slot],
                send_sem=send_k.at[h],
                recv_sem=recv_k.at[h],
                device_id=(right,),
                device_id_type=pl.DeviceIdType.MESH,
            )
            rv = pltpu.make_async_remote_copy(
                src_ref=vfull.at[slot],
                dst_ref=vfull.at[slot],
                send_sem=send_v.at[h],
                recv_sem=recv_v.at[h],
                device_id=(right,),
                device_id_type=pl.DeviceIdType.MESH,
            )
            rk.start()
            rv.start()
            rk.wait()
            rv.wait()

        row = lax.broadcasted_iota(jnp.int32, (SQ, N_DEV * SKV), 0)
        col = lax.broadcasted_iota(jnp.int32, (SQ, N_DEV * SKV), 1)
        qb = me * (SQ // BLK) + row // BLK
        kb = col // BLK
        mask = (qb == kb) | (kb == 0) | (lax.rem(qb + kb, 3) == 0)

        for b in range(B):
            qp = jnp.dot(x_ref[b], wq_ref[...],
                         preferred_element_type=jnp.float32)
            for h in range(HQ):
                q = qp[:, h * DH:(h + 1) * DH]
                parts = []
                for o in range(N_DEV):
                    k_o = kfull[o, b][:, h * DH:(h + 1) * DH]
                    parts.append(lax.dot_general(
                        q, k_o, (((1,), (1,)), ((), ())),
                        preferred_element_type=jnp.float32))
                s = jnp.concatenate(parts, axis=1) * 0.125
                s = jnp.where(mask, s, NEG)
                m = jnp.max(s, axis=1, keepdims=True)
                w = jnp.exp(s - m)
                w = w / jnp.sum(w, axis=1, keepdims=True)
                acc = jnp.zeros((SQ, DH), jnp.float32)
                for o in range(N_DEV):
                    v_o = vfull[o, b][:, h * DH:(h + 1) * DH]
                    acc = acc + jnp.dot(
                        w[:, o * SKV:(o + 1) * SKV], v_o,
                        preferred_element_type=jnp.float32)
                ctx[b, :, h * DH:(h + 1) * DH] = acc
            out_ref[b] = jnp.dot(ctx[b], wo_ref[...],
                                 preferred_element_type=jnp.float32)

    return pl.pallas_call(
        body,
        out_shape=jax.ShapeDtypeStruct((B, SQ, D_MODEL), jnp.float32),
        in_specs=[pl.BlockSpec(memory_space=pltpu.VMEM)] * 5,
        out_specs=pl.BlockSpec(memory_space=pltpu.VMEM),
        scratch_shapes=[
            pltpu.VMEM((N_DEV, B, SKV, D_QK), jnp.float32),
            pltpu.VMEM((N_DEV, B, SKV, D_QK), jnp.float32),
            pltpu.VMEM((B, SQ, D_QK), jnp.float32),
            pltpu.SemaphoreType.DMA((N_DEV - 1,)),
            pltpu.SemaphoreType.DMA((N_DEV - 1,)),
            pltpu.SemaphoreType.DMA((N_DEV - 1,)),
            pltpu.SemaphoreType.DMA((N_DEV - 1,)),
        ],
    )(x, Wq, k2, v2, Wo)


# device time: 112628 ns/iter; 1.7114x vs baseline; 1.7114x over previous
import jax
import jax.numpy as jnp
from jax import lax
from jax.experimental import pallas as pl
from jax.experimental.pallas import tpu as pltpu

N_DEV = 4
B = 2
SQ = 512
SKV = 512
HQ = 8
DH = 64
D_MODEL = 768
D_QK = HQ * DH
BLK = 64
HALF = SKV // 2
NEG = -1e9
F32 = jnp.float32


def kernel(x, Wq, K_ext, V_ext, Wo):
    k2 = K_ext.reshape(B, SKV, D_QK)
    v2 = V_ext.reshape(B, SKV, D_QK)

    def body(x_ref, wq_ref, k_ref, v_ref, wo_ref, out_ref,
             kfull, vfull, send_sems, recv_sems):
        me = lax.axis_index("i")
        right = lax.rem(me + 1, N_DEV)
        left = lax.rem(me + 3, N_DEV)
        opp = lax.rem(me + 2, N_DEV)

        def rdma(src, dst, i, dev):
            return pltpu.make_async_remote_copy(
                src_ref=src, dst_ref=dst,
                send_sem=send_sems.at[i], recv_sem=recv_sems.at[i],
                device_id=(dev,), device_id_type=pl.DeviceIdType.MESH,
            )

        h1 = [
            rdma(k_ref, kfull.at[me], 0, right),
            rdma(v_ref, vfull.at[me], 1, right),
            rdma(k_ref, kfull.at[me], 2, left),
            rdma(v_ref, vfull.at[me], 3, left),
        ]
        for r in h1:
            r.start()

        qp = [jnp.dot(x_ref[b], wq_ref[...], preferred_element_type=F32)
              for b in range(B)]

        def chunk_mask(origin, row_off, rows):
            r = lax.broadcasted_iota(jnp.int32, (SQ, rows), 0)
            c = lax.broadcasted_iota(jnp.int32, (SQ, rows), 1)
            qb = me * (SQ // BLK) + r // BLK
            kb = origin * (SKV // BLK) + (row_off + c) // BLK
            return (qb == kb) | (kb == 0) | (lax.rem(qb + kb, 3) == 0)

        state = {}

        def process(origin, row_off, rows, k_of_b, v_of_b):
            mask = chunk_mask(origin, row_off, rows)
            for b in range(B):
                kc = k_of_b(b)
                vc = v_of_b(b)
                for h in range(HQ):
                    q = qp[b][:, h * DH:(h + 1) * DH]
                    k_o = kc[:, h * DH:(h + 1) * DH]
                    v_o = vc[:, h * DH:(h + 1) * DH]
                    s = lax.dot_general(
                        q, k_o, (((1,), (1,)), ((), ())),
                        preferred_element_type=F32) * 0.125
                    s = jnp.where(mask, s, NEG)
                    m_c = jnp.max(s, axis=1, keepdims=True)
                    if (b, h) not in state:
                        p = jnp.exp(s - m_c)
                        d = jnp.sum(p, axis=1, keepdims=True)
                        acc = jnp.dot(p, v_o, preferred_element_type=F32)
                        state[(b, h)] = (m_c, d, acc)
                    else:
                        m0, d0, a0 = state[(b, h)]
                        m_n = jnp.maximum(m0, m_c)
                        alpha = jnp.exp(m0 - m_n)
                        p = jnp.exp(s - m_n)
                        d = d0 * alpha + jnp.sum(p, axis=1, keepdims=True)
                        acc = a0 * alpha + jnp.dot(
                            p, v_o, preferred_element_type=F32)
                        state[(b, h)] = (m_n, d, acc)

        process(me, 0, SKV, lambda b: k_ref[b], lambda b: v_ref[b])

        h1[0].wait_recv()
        h1[1].wait_recv()
        h2r = [
            rdma(kfull.at[left, :, pl.ds(0, HALF), :],
                 kfull.at[left, :, pl.ds(0, HALF), :], 4, right),
            rdma(vfull.at[left, :, pl.ds(0, HALF), :],
                 vfull.at[left, :, pl.ds(0, HALF), :], 5, right),
        ]
        for r in h2r:
            r.start()
        process(left, 0, SKV,
                lambda b: kfull[left, b], lambda b: vfull[left, b])

        h1[2].wait_recv()
        h1[3].wait_recv()
        h2l = [
            rdma(kfull.at[right, :, pl.ds(HALF, HALF), :],
                 kfull.at[right, :, pl.ds(HALF, HALF), :], 6, left),
            rdma(vfull.at[right, :, pl.ds(HALF, HALF), :],
                 vfull.at[right, :, pl.ds(HALF, HALF), :], 7, left),
        ]
        for r in h2l:
            r.start()
        process(right, 0, SKV,
                lambda b: kfull[right, b], lambda b: vfull[right, b])

        for r in h2r + h2l:
            r.wait_recv()
        process(opp, 0, SKV,
                lambda b: kfull[opp, b], lambda b: vfull[opp, b])

        for b in range(B):
            ctx = jnp.concatenate(
                [state[(b, h)][2] / state[(b, h)][1] for h in range(HQ)],
                axis=1)
            out_ref[b] = jnp.dot(ctx, wo_ref[...],
                                 preferred_element_type=F32)

        for r in h1 + h2r + h2l:
            r.wait_send()

    return pl.pallas_call(
        body,
        out_shape=jax.ShapeDtypeStruct((B, SQ, D_MODEL), jnp.float32),
        in_specs=[pl.BlockSpec(memory_space=pltpu.VMEM)] * 5,
        out_specs=pl.BlockSpec(memory_space=pltpu.VMEM),
        scratch_shapes=[
            pltpu.VMEM((N_DEV, B, SKV, D_QK), jnp.float32),
            pltpu.VMEM((N_DEV, B, SKV, D_QK), jnp.float32),
            pltpu.SemaphoreType.DMA((8,)),
            pltpu.SemaphoreType.DMA((8,)),
        ],
        compiler_params=pltpu.CompilerParams(
            vmem_limit_bytes=100 * 1024 * 1024,
        ),
    )(x, Wq, k2, v2, Wo)


# device time: 82006 ns/iter; 2.3505x vs baseline; 1.3734x over previous
import jax
import jax.numpy as jnp
from jax import lax
from jax.experimental import pallas as pl
from jax.experimental.pallas import tpu as pltpu

N_DEV = 4
B = 2
SQ = 512
SKV = 512
HQ = 8
DH = 64
D_MODEL = 768
D_QK = HQ * DH
BLK = 64
HALF = SKV // 2
NEG = -1e9
F32 = jnp.float32
BF16 = jnp.bfloat16


def kernel(x, Wq, K_ext, V_ext, Wo):
    k2 = K_ext.reshape(B, SKV, D_QK)
    v2 = V_ext.reshape(B, SKV, D_QK)

    def body(x_ref, wq_ref, k_ref, v_ref, wo_ref, out_ref,
             kfull, vfull, send_sems, recv_sems):
        me = lax.axis_index("i")
        right = lax.rem(me + 1, N_DEV)
        left = lax.rem(me + 3, N_DEV)
        opp = lax.rem(me + 2, N_DEV)

        def rdma(src, dst, i, dev):
            return pltpu.make_async_remote_copy(
                src_ref=src, dst_ref=dst,
                send_sem=send_sems.at[i], recv_sem=recv_sems.at[i],
                device_id=(dev,), device_id_type=pl.DeviceIdType.MESH,
            )

        kfull[me] = k_ref[...].astype(BF16)
        vfull[me] = v_ref[...].astype(BF16)

        h1 = [
            rdma(kfull.at[me], kfull.at[me], 0, right),
            rdma(vfull.at[me], vfull.at[me], 1, right),
            rdma(kfull.at[me], kfull.at[me], 2, left),
            rdma(vfull.at[me], vfull.at[me], 3, left),
        ]
        for r in h1:
            r.start()

        qp = [jnp.dot(x_ref[b].astype(BF16), wq_ref[...].astype(BF16),
                      preferred_element_type=F32).astype(BF16)
              for b in range(B)]

        def chunk_mask(origin, rows):
            r = lax.broadcasted_iota(jnp.int32, (SQ, rows), 0)
            c = lax.broadcasted_iota(jnp.int32, (SQ, rows), 1)
            qb = me * (SQ // BLK) + r // BLK
            kb = origin * (SKV // BLK) + c // BLK
            return (qb == kb) | (kb == 0) | (lax.rem(qb + kb, 3) == 0)

        state = {}

        def process(origin):
            mask = chunk_mask(origin, SKV)
            for b in range(B):
                kc = kfull[origin, b]
                vc = vfull[origin, b]
                for h in range(HQ):
                    q = qp[b][:, h * DH:(h + 1) * DH]
                    k_o = kc[:, h * DH:(h + 1) * DH]
                    v_o = vc[:, h * DH:(h + 1) * DH]
                    s = lax.dot_general(
                        q, k_o, (((1,), (1,)), ((), ())),
                        preferred_element_type=F32) * 0.125
                    s = jnp.where(mask, s, NEG)
                    m_c = jnp.max(s, axis=1, keepdims=True)
                    if (b, h) not in state:
                        p = jnp.exp(s - m_c)
                        d = jnp.sum(p, axis=1, keepdims=True)
                        acc = jnp.dot(p.astype(BF16), v_o,
                                      preferred_element_type=F32)
                        state[(b, h)] = (m_c, d, acc)
                    else:
                        m0, d0, a0 = state[(b, h)]
                        m_n = jnp.maximum(m0, m_c)
                        alpha = jnp.exp(m0 - m_n)
                        p = jnp.exp(s - m_n)
                        d = d0 * alpha + jnp.sum(p, axis=1, keepdims=True)
                        acc = a0 * alpha + jnp.dot(
                            p.astype(BF16), v_o,
                            preferred_element_type=F32)
                        state[(b, h)] = (m_n, d, acc)

        process(me)

        h1[0].wait_recv()
        h1[1].wait_recv()
        h2r = [
            rdma(kfull.at[left, :, pl.ds(0, HALF), :],
                 kfull.at[left, :, pl.ds(0, HALF), :], 4, right),
            rdma(vfull.at[left, :, pl.ds(0, HALF), :],
                 vfull.at[left, :, pl.ds(0, HALF), :], 5, right),
        ]
        for r in h2r:
            r.start()
        process(left)

        h1[2].wait_recv()
        h1[3].wait_recv()
        h2l = [
            rdma(kfull.at[right, :, pl.ds(HALF, HALF), :],
                 kfull.at[right, :, pl.ds(HALF, HALF), :], 6, left),
            rdma(vfull.at[right, :, pl.ds(HALF, HALF), :],
                 vfull.at[right, :, pl.ds(HALF, HALF), :], 7, left),
        ]
        for r in h2l:
            r.start()
        process(right)

        for r in h2r + h2l:
            r.wait_recv()
        process(opp)

        wo_b = wo_ref[...].astype(BF16)
        for b in range(B):
            ctx = jnp.concatenate(
                [state[(b, h)][2] / state[(b, h)][1] for h in range(HQ)],
                axis=1)
            out_ref[b] = jnp.dot(ctx.astype(BF16), wo_b,
                                 preferred_element_type=F32)

        for r in h1 + h2r + h2l:
            r.wait_send()

    return pl.pallas_call(
        body,
        out_shape=jax.ShapeDtypeStruct((B, SQ, D_MODEL), jnp.float32),
        in_specs=[pl.BlockSpec(memory_space=pltpu.VMEM)] * 5,
        out_specs=pl.BlockSpec(memory_space=pltpu.VMEM),
        scratch_shapes=[
            pltpu.VMEM((N_DEV, B, SKV, D_QK), BF16),
            pltpu.VMEM((N_DEV, B, SKV, D_QK), BF16),
            pltpu.SemaphoreType.DMA((8,)),
            pltpu.SemaphoreType.DMA((8,)),
        ],
        compiler_params=pltpu.CompilerParams(
            vmem_limit_bytes=100 * 1024 * 1024,
        ),
    )(x, Wq, k2, v2, Wo)


# device time: 71595 ns/iter; 2.6923x vs baseline; 1.1454x over previous
import jax
import jax.numpy as jnp
from jax import lax
from jax.experimental import pallas as pl
from jax.experimental.pallas import tpu as pltpu

N_DEV = 4
B = 2
SQ = 512
SKV = 512
HQ = 8
DH = 64
D_MODEL = 768
D_QK = HQ * DH
BLK = 64
HALF = SKV // 2
NEG = -1e9
F32 = jnp.float32
BF16 = jnp.bfloat16
LOG2E = 1.4426950408889634


def kernel(x, Wq, K_ext, V_ext, Wo):
    k2 = K_ext.reshape(B, SKV, D_QK)
    v2 = V_ext.reshape(B, SKV, D_QK)

    def body(x_ref, wq_ref, k_ref, v_ref, wo_ref, out_ref,
             kfull, vfull, send_sems, recv_sems):
        me = lax.axis_index("i")
        right = lax.rem(me + 1, N_DEV)
        left = lax.rem(me + 3, N_DEV)
        opp = lax.rem(me + 2, N_DEV)

        def rdma(src, dst, i, dev):
            return pltpu.make_async_remote_copy(
                src_ref=src, dst_ref=dst,
                send_sem=send_sems.at[i], recv_sem=recv_sems.at[i],
                device_id=(dev,), device_id_type=pl.DeviceIdType.MESH,
            )

        kfull[me] = k_ref[...].astype(BF16)
        h1 = [
            rdma(kfull.at[me], kfull.at[me], 0, right),
            rdma(kfull.at[me], kfull.at[me], 2, left),
        ]
        h1[0].start()
        h1[1].start()
        vfull[me] = v_ref[...].astype(BF16)
        h1 += [
            rdma(vfull.at[me], vfull.at[me], 1, right),
            rdma(vfull.at[me], vfull.at[me], 3, left),
        ]
        h1[2].start()
        h1[3].start()

        wq16 = (wq_ref[...] * (0.125 * LOG2E)).astype(BF16)
        qp = [jnp.dot(x_ref[b].astype(BF16), wq16,
                      preferred_element_type=F32).astype(BF16)
              for b in range(B)]

        def chunk_mask(origin):
            r = lax.broadcasted_iota(jnp.int32, (SQ, SKV), 0)
            c = lax.broadcasted_iota(jnp.int32, (SQ, SKV), 1)
            qb = me * (SQ // BLK) + r // BLK
            kb = origin * (SKV // BLK) + c // BLK
            return (qb == kb) | (kb == 0) | (lax.rem(qb + kb, 3) == 0)

        state = {}

        def process(origin):
            mask = chunk_mask(origin)
            for b in range(B):
                kc = kfull[origin, b]
                vc = vfull[origin, b]
                for h in range(HQ):
                    q = qp[b][:, h * DH:(h + 1) * DH]
                    k_o = kc[:, h * DH:(h + 1) * DH]
                    v_o = vc[:, h * DH:(h + 1) * DH]
                    s = lax.dot_general(
                        q, k_o, (((1,), (1,)), ((), ())),
                        preferred_element_type=F32)
                    w = jnp.exp2(jnp.where(mask, s, NEG))
                    d = jnp.sum(w, axis=1, keepdims=True)
                    acc = jnp.dot(w.astype(BF16), v_o,
                                  preferred_element_type=F32)
                    if (b, h) not in state:
                        state[(b, h)] = (d, acc)
                    else:
                        d0, a0 = state[(b, h)]
                        state[(b, h)] = (d0 + d, a0 + acc)

        process(me)

        h1[0].wait_recv()
        h1[2].wait_recv()
        h2r = [
            rdma(kfull.at[left, :, pl.ds(0, HALF), :],
                 kfull.at[left, :, pl.ds(0, HALF), :], 4, right),
            rdma(vfull.at[left, :, pl.ds(0, HALF), :],
                 vfull.at[left, :, pl.ds(0, HALF), :], 5, right),
        ]
        for r in h2r:
            r.start()
        process(left)

        h1[1].wait_recv()
        h1[3].wait_recv()
        h2l = [
            rdma(kfull.at[right, :, pl.ds(HALF, HALF), :],
                 kfull.at[right, :, pl.ds(HALF, HALF), :], 6, left),
            rdma(vfull.at[right, :, pl.ds(HALF, HALF), :],
                 vfull.at[right, :, pl.ds(HALF, HALF), :], 7, left),
        ]
        for r in h2l:
            r.start()
        process(right)

        for r in h2r + h2l:
            r.wait_recv()
        process(opp)

        wo_b = wo_ref[...].astype(BF16)
        for b in range(B):
            ctx = jnp.concatenate(
                [state[(b, h)][1] / state[(b, h)][0] for h in range(HQ)],
                axis=1)
            out_ref[b] = jnp.dot(ctx.astype(BF16), wo_b,
                                 preferred_element_type=F32)

        for r in h1 + h2r + h2l:
            r.wait_send()

    return pl.pallas_call(
        body,
        out_shape=jax.ShapeDtypeStruct((B, SQ, D_MODEL), jnp.float32),
        in_specs=[pl.BlockSpec(memory_space=pltpu.VMEM)] * 5,
        out_specs=pl.BlockSpec(memory_space=pltpu.VMEM),
        scratch_shapes=[
            pltpu.VMEM((N_DEV, B, SKV, D_QK), BF16),
            pltpu.VMEM((N_DEV, B, SKV, D_QK), BF16),
            pltpu.SemaphoreType.DMA((8,)),
            pltpu.SemaphoreType.DMA((8,)),
        ],
        compiler_params=pltpu.CompilerParams(
            vmem_limit_bytes=100 * 1024 * 1024,
        ),
    )(x, Wq, k2, v2, Wo)


# device time: 60815 ns/iter; 3.1696x vs baseline; 1.1773x over previous
import jax
import jax.numpy as jnp
from jax import lax
from jax.experimental import pallas as pl
from jax.experimental.pallas import tpu as pltpu

N_DEV = 4
B = 2
SQ = 512
SKV = 512
HQ = 8
DH = 64
D_MODEL = 768
D_QK = HQ * DH
BLK = 64
HALF = SKV // 2
NEG = -1e9
F32 = jnp.float32
BF16 = jnp.bfloat16
LOG2E = 1.4426950408889634


def kernel(x, Wq, K_ext, V_ext, Wo):
    k2 = K_ext.reshape(B, SKV, D_QK)
    v2 = V_ext.reshape(B, SKV, D_QK)

    def body(x_ref, wq_ref, k_ref, v_ref, wo_ref, out_ref,
             kfull, vfull, send_sems, recv_sems):
        me = lax.axis_index("i")
        right = lax.rem(me + 1, N_DEV)
        left = lax.rem(me + 3, N_DEV)
        opp = lax.rem(me + 2, N_DEV)

        def rdma(src, dst, i, dev):
            return pltpu.make_async_remote_copy(
                src_ref=src, dst_ref=dst,
                send_sem=send_sems.at[i], recv_sem=recv_sems.at[i],
                device_id=(dev,), device_id_type=pl.DeviceIdType.MESH,
            )

        barrier_sem = pltpu.get_barrier_semaphore()
        for nbr in (left, right):
            pl.semaphore_signal(
                barrier_sem, inc=1,
                device_id=(nbr,), device_id_type=pl.DeviceIdType.MESH,
            )
        pl.semaphore_wait(barrier_sem, 2)

        kfull[me] = k_ref[...].astype(BF16)
        h1 = [
            rdma(kfull.at[me], kfull.at[me], 0, right),
            rdma(kfull.at[me], kfull.at[me], 2, left),
        ]
        h1[0].start()
        h1[1].start()
        vfull[me] = v_ref[...].astype(BF16)
        h1 += [
            rdma(vfull.at[me], vfull.at[me], 1, right),
            rdma(vfull.at[me], vfull.at[me], 3, left),
        ]
        h1[2].start()
        h1[3].start()

        wq16 = (wq_ref[...] * (0.125 * LOG2E)).astype(BF16)
        qp = [jnp.dot(x_ref[b].astype(BF16), wq16,
                      preferred_element_type=F32).astype(BF16)
              for b in range(B)]

        def chunk_mask(origin):
            r = lax.broadcasted_iota(jnp.int32, (SQ, SKV), 0)
            c = lax.broadcasted_iota(jnp.int32, (SQ, SKV), 1)
            qb = me * (SQ // BLK) + r // BLK
            kb = origin * (SKV // BLK) + c // BLK
            return (qb == kb) | (kb == 0) | (lax.rem(qb + kb, 3) == 0)

        masks = {name: chunk_mask(org)
                 for name, org in (("me", me), ("left", left),
                                   ("right", right), ("opp", opp))}

        state = {}

        def process(origin, mask_name):
            mask = masks[mask_name]
            for b in range(B):
                kc = kfull[origin, b]
                vc = vfull[origin, b]
                for h in range(HQ):
                    q = qp[b][:, h * DH:(h + 1) * DH]
                    k_o = kc[:, h * DH:(h + 1) * DH]
                    v_o = vc[:, h * DH:(h + 1) * DH]
                    s = lax.dot_general(
                        q, k_o, (((1,), (1,)), ((), ())),
                        preferred_element_type=F32)
                    w = jnp.exp2(jnp.where(mask, s, NEG))
                    d = jnp.sum(w, axis=1, keepdims=True)
                    acc = jnp.dot(w.astype(BF16), v_o,
                                  preferred_element_type=F32)
                    if (b, h) not in state:
                        state[(b, h)] = (d, acc)
                    else:
                        d0, a0 = state[(b, h)]
                        state[(b, h)] = (d0 + d, a0 + acc)

        process(me, "me")

        h1[0].wait_recv()
        h1[2].wait_recv()
        h2r = [
            rdma(kfull.at[left, :, pl.ds(0, HALF), :],
                 kfull.at[left, :, pl.ds(0, HALF), :], 4, right),
            rdma(vfull.at[left, :, pl.ds(0, HALF), :],
                 vfull.at[left, :, pl.ds(0, HALF), :], 5, right),
        ]
        for r in h2r:
            r.start()
        h1[1].wait_recv()
        h1[3].wait_recv()
        h2l = [
            rdma(kfull.at[right, :, pl.ds(HALF, HALF), :],
                 kfull.at[right, :, pl.ds(HALF, HALF), :], 6, left),
            rdma(vfull.at[right, :, pl.ds(HALF, HALF), :],
                 vfull.at[right, :, pl.ds(HALF, HALF), :], 7, left),
        ]
        for r in h2l:
            r.start()

        process(left, "left")
        process(right, "right")

        for r in h2r + h2l:
            r.wait_recv()
        process(opp, "opp")

        wo_b = wo_ref[...].astype(BF16)
        for b in range(B):
            ctx = jnp.concatenate(
                [state[(b, h)][1] / state[(b, h)][0] for h in range(HQ)],
                axis=1)
            out_ref[b] = jnp.dot(ctx.astype(BF16), wo_b,
                                 preferred_element_type=F32)

        for r in h1 + h2r + h2l:
            r.wait_send()

    return pl.pallas_call(
        body,
        out_shape=jax.ShapeDtypeStruct((B, SQ, D_MODEL), jnp.float32),
        in_specs=[pl.BlockSpec(memory_space=pltpu.VMEM)] * 5,
        out_specs=pl.BlockSpec(memory_space=pltpu.VMEM),
        scratch_shapes=[
            pltpu.VMEM((N_DEV, B, SKV, D_QK), BF16),
            pltpu.VMEM((N_DEV, B, SKV, D_QK), BF16),
            pltpu.SemaphoreType.DMA((8,)),
            pltpu.SemaphoreType.DMA((8,)),
        ],
        compiler_params=pltpu.CompilerParams(
            vmem_limit_bytes=100 * 1024 * 1024,
            collective_id=0,
        ),
    )(x, Wq, k2, v2, Wo)


# device time: 54439 ns/iter; 3.5408x vs baseline; 1.1171x over previous
import jax
import jax.numpy as jnp
from jax import lax
from jax.experimental import pallas as pl
from jax.experimental.pallas import tpu as pltpu

N_DEV = 4
B = 2
SQ = 512
SKV = 512
HQ = 8
DH = 64
D_MODEL = 768
D_QK = HQ * DH
BLK = 64
HALF = SKV // 2
NEG = -1e9
F32 = jnp.float32
BF16 = jnp.bfloat16
FP8 = jnp.float8_e4m3fn
LOG2E = 1.4426950408889634


def kernel(x, Wq, K_ext, V_ext, Wo):
    k2 = K_ext.reshape(B, SKV, D_QK)
    v2 = V_ext.reshape(B, SKV, D_QK)

    def body(x_ref, wq_ref, k_ref, v_ref, wo_ref, out_ref,
             kfull, vfull, send_sems, recv_sems):
        me = lax.axis_index("i")
        right = lax.rem(me + 1, N_DEV)
        left = lax.rem(me + 3, N_DEV)
        opp = lax.rem(me + 2, N_DEV)

        def rdma(src, dst, i, dev):
            return pltpu.make_async_remote_copy(
                src_ref=src, dst_ref=dst,
                send_sem=send_sems.at[i], recv_sem=recv_sems.at[i],
                device_id=(dev,), device_id_type=pl.DeviceIdType.MESH,
            )

        barrier_sem = pltpu.get_barrier_semaphore()
        for nbr in (left, right):
            pl.semaphore_signal(
                barrier_sem, inc=1,
                device_id=(nbr,), device_id_type=pl.DeviceIdType.MESH,
            )
        pl.semaphore_wait(barrier_sem, 2)

        kfull[me] = k_ref[...].astype(FP8)
        h1 = [
            rdma(kfull.at[me], kfull.at[me], 0, right),
            rdma(kfull.at[me], kfull.at[me], 2, left),
        ]
        h1[0].start()
        h1[1].start()
        vfull[me] = v_ref[...].astype(BF16)
        h1 += [
            rdma(vfull.at[me], vfull.at[me], 1, right),
            rdma(vfull.at[me], vfull.at[me], 3, left),
        ]
        h1[2].start()
        h1[3].start()

        wq16 = (wq_ref[...] * (0.125 * LOG2E)).astype(BF16)
        qp = [jnp.dot(x_ref[b].astype(BF16), wq16,
                      preferred_element_type=F32).astype(BF16)
              for b in range(B)]

        def chunk_mask(origin):
            r = lax.broadcasted_iota(jnp.int32, (SQ, SKV), 0)
            c = lax.broadcasted_iota(jnp.int32, (SQ, SKV), 1)
            qb = me * (SQ // BLK) + r // BLK
            kb = origin * (SKV // BLK) + c // BLK
            return (qb == kb) | (kb == 0) | (lax.rem(qb + kb, 3) == 0)

        masks = {name: chunk_mask(org)
                 for name, org in (("me", me), ("left", left),
                                   ("right", right), ("opp", opp))}

        state = {}
        ones_col = jnp.ones((SKV, 1), BF16)

        def process(origin, mask_name, kc_fn, vc_fn):
            mask = masks[mask_name]
            for b in range(B):
                kc = kc_fn(b)
                vc = vc_fn(b)
                for h in range(HQ):
                    q = qp[b][:, h * DH:(h + 1) * DH]
                    k_o = kc[:, h * DH:(h + 1) * DH]
                    v_aug = jnp.concatenate(
                        [vc[:, h * DH:(h + 1) * DH], ones_col], axis=1)
                    s = lax.dot_general(
                        q, k_o, (((1,), (1,)), ((), ())),
                        preferred_element_type=F32)
                    w = jnp.exp2(jnp.where(mask, s, NEG))
                    aug = jnp.dot(w.astype(BF16), v_aug,
                                  preferred_element_type=F32)
                    if (b, h) not in state:
                        state[(b, h)] = aug
                    else:
                        state[(b, h)] = state[(b, h)] + aug

        process(me, "me",
                lambda b: k_ref[b].astype(BF16),
                lambda b: vfull[me, b])

        h1[0].wait_recv()
        h1[2].wait_recv()
        h2r = [
            rdma(kfull.at[left, :, pl.ds(0, HALF), :],
                 kfull.at[left, :, pl.ds(0, HALF), :], 4, right),
            rdma(vfull.at[left, :, pl.ds(0, HALF), :],
                 vfull.at[left, :, pl.ds(0, HALF), :], 5, right),
        ]
        for r in h2r:
            r.start()
        h1[1].wait_recv()
        h1[3].wait_recv()
        h2l = [
            rdma(kfull.at[right, :, pl.ds(HALF, HALF), :],
                 kfull.at[right, :, pl.ds(HALF, HALF), :], 6, left),
            rdma(vfull.at[right, :, pl.ds(HALF, HALF), :],
                 vfull.at[right, :, pl.ds(HALF, HALF), :], 7, left),
        ]
        for r in h2l:
            r.start()

        def remote(origin):
            return (lambda b: kfull[origin, b].astype(BF16),
                    lambda b: vfull[origin, b])

        process(left, "left", *remote(left))
        process(right, "right", *remote(right))

        for r in h2r + h2l:
            r.wait_recv()
        process(opp, "opp", *remote(opp))

        wo_b = wo_ref[...].astype(BF16)
        for b in range(B):
            ctx = jnp.concatenate(
                [state[(b, h)][:, :DH] / state[(b, h)][:, DH:DH + 1]
                 for h in range(HQ)],
                axis=1)
            out_ref[b] = jnp.dot(ctx.astype(BF16), wo_b,
                                 preferred_element_type=F32)

        for r in h1 + h2r + h2l:
            r.wait_send()

    return pl.pallas_call(
        body,
        out_shape=jax.ShapeDtypeStruct((B, SQ, D_MODEL), jnp.float32),
        in_specs=[pl.BlockSpec(memory_space=pltpu.VMEM)] * 5,
        out_specs=pl.BlockSpec(memory_space=pltpu.VMEM),
        scratch_shapes=[
            pltpu.VMEM((N_DEV, B, SKV, D_QK), FP8),
            pltpu.VMEM((N_DEV, B, SKV, D_QK), BF16),
            pltpu.SemaphoreType.DMA((8,)),
            pltpu.SemaphoreType.DMA((8,)),
        ],
        compiler_params=pltpu.CompilerParams(
            vmem_limit_bytes=100 * 1024 * 1024,
            collective_id=0,
        ),
    )(x, Wq, k2, v2, Wo)


# device time: 50797 ns/iter; 3.7947x vs baseline; 1.0717x over previous
import jax
import jax.numpy as jnp
from jax import lax
from jax.experimental import pallas as pl
from jax.experimental.pallas import tpu as pltpu

N_DEV = 4
B = 2
SQ = 512
SKV = 512
HQ = 8
DH = 64
D_MODEL = 768
D_QK = HQ * DH
BLK = 64
HALF = SKV // 2
NEG = -1e9
F32 = jnp.float32
BF16 = jnp.bfloat16
FP8 = jnp.float8_e4m3fn
LOG2E = 1.4426950408889634

S_K_R, S_V0_R, S_V1_R = 0, 1, 2
S_K_L, S_V1_L, S_V0_L = 3, 4, 5
S_K_H0, S_V_H0 = 6, 7
S_K_H1, S_V_H1 = 8, 9


def kernel(x, Wq, K_ext, V_ext, Wo):
    k2 = K_ext.reshape(B, SKV, D_QK)
    v2 = V_ext.reshape(B, SKV, D_QK)

    def body(x_ref, wq_ref, k_ref, v_ref, wo_ref, out_ref,
             kfull, vfull, send_sems, recv_sems):
        me = lax.axis_index("i")
        right = lax.rem(me + 1, N_DEV)
        left = lax.rem(me + 3, N_DEV)
        opp = lax.rem(me + 2, N_DEV)

        def rdma(src, dst, i, dev):
            return pltpu.make_async_remote_copy(
                src_ref=src, dst_ref=dst,
                send_sem=send_sems.at[i], recv_sem=recv_sems.at[i],
                device_id=(dev,), device_id_type=pl.DeviceIdType.MESH,
            )

        barrier_sem = pltpu.get_barrier_semaphore()
        for nbr in (left, right):
            pl.semaphore_signal(
                barrier_sem, inc=1,
                device_id=(nbr,), device_id_type=pl.DeviceIdType.MESH,
            )
        pl.semaphore_wait(barrier_sem, 2)

        def vh(buf, slot, half):
            return buf.at[slot, :, pl.ds(half * HALF, HALF), :]

        kfull[me] = k_ref[...].astype(FP8)
        xfers = {
            S_K_R: rdma(kfull.at[me], kfull.at[me], S_K_R, right),
            S_K_L: rdma(kfull.at[me], kfull.at[me], S_K_L, left),
        }
        xfers[S_K_R].start()
        xfers[S_K_L].start()
        vfull[me] = v_ref[...].astype(BF16)
        for i, slot_half, dev in (
            (S_V0_R, 0, right), (S_V1_L, 1, left),
            (S_V1_R, 1, right), (S_V0_L, 0, left),
        ):
            xfers[i] = rdma(vh(vfull, me, slot_half),
                            vh(vfull, me, slot_half), i, dev)
            xfers[i].start()

        wq16 = (wq_ref[...] * (0.125 * LOG2E)).astype(BF16)
        qp = [jnp.dot(x_ref[b].astype(BF16), wq16,
                      preferred_element_type=F32).astype(BF16)
              for b in range(B)]

        def chunk_mask(origin):
            r = lax.broadcasted_iota(jnp.int32, (SQ, SKV), 0)
            c = lax.broadcasted_iota(jnp.int32, (SQ, SKV), 1)
            qb = me * (SQ // BLK) + r // BLK
            kb = origin * (SKV // BLK) + c // BLK
            return (qb == kb) | (kb == 0) | (lax.rem(qb + kb, 3) == 0)

        masks = {name: chunk_mask(org)
                 for name, org in (("me", me), ("left", left),
                                   ("right", right), ("opp", opp))}

        state = {}
        ones_col = jnp.ones((SKV, 1), BF16)

        def process(mask_name, kc_fn, vc_fn, half=None):
            lo, sz = (0, SKV) if half is None else (half * HALF, HALF)
            mask = masks[mask_name][:, lo:lo + sz]
            for b in range(B):
                kc = kc_fn(b)[lo:lo + sz, :]
                vc = vc_fn(b)[lo:lo + sz, :]
                for h in range(HQ):
                    q = qp[b][:, h * DH:(h + 1) * DH]
                    k_o = kc[:, h * DH:(h + 1) * DH]
                    v_aug = jnp.concatenate(
                        [vc[:, h * DH:(h + 1) * DH], ones_col[:sz]],
                        axis=1)
                    s = lax.dot_general(
                        q, k_o, (((1,), (1,)), ((), ())),
                        preferred_element_type=F32)
                    w = jnp.exp2(jnp.where(mask, s, NEG))
                    aug = jnp.dot(w.astype(BF16), v_aug,
                                  preferred_element_type=F32)
                    if (b, h) not in state:
                        state[(b, h)] = aug
                    else:
                        state[(b, h)] = state[(b, h)] + aug

        def remote(origin):
            return (lambda b: kfull[origin, b].astype(BF16),
                    lambda b: vfull[origin, b])

        process("me",
                lambda b: k_ref[b].astype(BF16),
                lambda b: vfull[me, b])

        xfers[S_K_R].wait_recv()
        xfers[S_V0_R].wait_recv()
        h2r = [rdma(vh(kfull, left, 0), vh(kfull, left, 0),
                    S_K_H0, right),
               rdma(vh(vfull, left, 0), vh(vfull, left, 0),
                    S_V_H0, right)]
        for r in h2r:
            r.start()
        xfers[S_K_L].wait_recv()
        xfers[S_V1_L].wait_recv()
        h2l = [rdma(vh(kfull, right, 1), vh(kfull, right, 1),
                    S_K_H1, left),
               rdma(vh(vfull, right, 1), vh(vfull, right, 1),
                    S_V_H1, left)]
        for r in h2l:
            r.start()

        process("left", *remote(left), half=0)
        xfers[S_V1_R].wait_recv()
        process("left", *remote(left), half=1)
        process("right", *remote(right), half=1)
        xfers[S_V0_L].wait_recv()
        process("right", *remote(right), half=0)

        for r in h2r + h2l:
            r.wait_recv()
        process("opp", *remote(opp))

        wo_b = wo_ref[...].astype(BF16)
        for b in range(B):
            ctx = jnp.concatenate(
                [state[(b, h)][:, :DH] / state[(b, h)][:, DH:DH + 1]
                 for h in range(HQ)],
                axis=1)
            out_ref[b] = jnp.dot(ctx.astype(BF16), wo_b,
                                 preferred_element_type=F32)

        for r in list(xfers.values()) + h2r + h2l:
            r.wait_send()

    return pl.pallas_call(
        body,
        out_shape=jax.ShapeDtypeStruct((B, SQ, D_MODEL), jnp.float32),
        in_specs=[pl.BlockSpec(memory_space=pltpu.VMEM)] * 5,
        out_specs=pl.BlockSpec(memory_space=pltpu.VMEM),
        scratch_shapes=[
            pltpu.VMEM((N_DEV, B, SKV, D_QK), FP8),
            pltpu.VMEM((N_DEV, B, SKV, D_QK), BF16),
            pltpu.SemaphoreType.DMA((10,)),
            pltpu.SemaphoreType.DMA((10,)),
        ],
        compiler_params=pltpu.CompilerParams(
            vmem_limit_bytes=100 * 1024 * 1024,
            collective_id=0,
        ),
    )(x, Wq, k2, v2, Wo)
